# baseline (speedup 1.0000x reference)
import jax
import jax.numpy as jnp
from jax import lax
from jax.experimental import pallas as pl
from jax.experimental.pallas import tpu as pltpu
from jax.experimental.pallas import tpu_sc as plsc

TOPK = 5
IGN = -100
V = 100000
N = 1024
L = 16
NW = 32
ROWS_PER_W = N // NW
C = V // 2
CV = C // L
UNROLL = 8


def _body(flat_hbm, targ_hbm, out_hbm, targv, idxv, tvals, buf0, buf1, resv,
          sem_g, sem_c):
    wid = lax.axis_index("s") * 2 + lax.axis_index("c")
    base = wid * ROWS_PER_W

    pltpu.async_copy(flat_hbm.at[pl.ds(base * V, C)], buf0, sem_c)

    pltpu.sync_copy(targ_hbm.at[pl.ds(base, ROWS_PER_W)], targv)
    lane = lax.iota(jnp.int32, L)
    for jj in range(ROWS_PER_W // L):
        t16 = targv[pl.ds(jj * L, L)]
        rows16 = jnp.full((L,), base + jj * L, jnp.int32) + lane
        idxv[pl.ds(jj * L, L)] = rows16 * V + jnp.maximum(t16, 0)
    pltpu.async_copy(flat_hbm.at[idxv], tvals, sem_g).wait()

    bufs = (buf0, buf1)
    zero16 = jnp.zeros((L,), jnp.int32)

    def row_body(j, carry):
        cc, vc = carry
        row = base + j
        j16 = jnp.full((L,), j, jnp.int32)
        tidx16 = plsc.load_gather(targv, [j16])
        tval16 = plsc.load_gather(tvals, [j16])
        t = jnp.max(tidx16)
        tc = jnp.clip(t, 0, V - 1)

        acc = zero16
        pcnt = zero16
        for c in range(2):
            buf = bufs[c]
            o = c * C
            pltpu.make_async_copy(flat_hbm.at[pl.ds(0, C)], buf, sem_c).wait()
            nxt = jnp.where(c == 0, row * V + C, (row + 1) * V)
            nxt = jnp.minimum(nxt, N * V - C)
            pltpu.async_copy(flat_hbm.at[pl.ds(nxt, C)], bufs[1 - c], sem_c)

            s = jnp.clip(tc - o, 0, C)
            fs = s // L

            @plsc.parallel_loop(0, fs, unroll=UNROLL, carry=pcnt)
            def ge_loop(i, a):
                x = buf[pl.ds(i * L, L)]
                return a + plsc.all_reduce_population_count(x >= tval16)
            pcnt = ge_loop

            fm = jnp.minimum(fs, CV - 1)
            x = buf[pl.ds(fm * L, L)]
            posv = jnp.full((L,), o + fm * L, jnp.int32) + lane
            m = (x > tval16) | ((x == tval16) & (posv < tidx16))
            m = m & (jnp.full((L,), fs, jnp.int32) < CV)
            acc = acc + jnp.where(m, 1, 0).astype(jnp.int32)

            @plsc.parallel_loop(fs + 1, CV, unroll=UNROLL, carry=pcnt)
            def gt_loop(i, a):
                x = buf[pl.ds(i * L, L)]
                return a + plsc.all_reduce_population_count(x > tval16)
            pcnt = gt_loop

        rank = jnp.sum(acc) + jnp.max(pcnt)
        valid = t != IGN
        hit = (rank < TOPK) & valid
        cc = cc + jnp.where(hit, 1.0, 0.0)
        vc = vc + jnp.where(valid, 1.0, 0.0)
        return cc, vc

    cc, vc = lax.fori_loop(0, ROWS_PER_W, row_body,
                           (jnp.float32(0.0), jnp.float32(0.0)))
    pltpu.make_async_copy(flat_hbm.at[pl.ds(0, C)], buf0, sem_c).wait()

    resv[...] = jnp.where(lane == 0, cc, jnp.where(lane == 1, vc, 0.0))
    pltpu.sync_copy(resv, out_hbm.at[wid])


@jax.jit
def kernel(logits, targets):
    flat = logits.reshape(-1)
    tflat = targets.reshape(-1).astype(jnp.int32)
    mesh = plsc.VectorSubcoreMesh(core_axis_name="c", subcore_axis_name="s")
    out = pl.kernel(
        _body,
        out_type=jax.ShapeDtypeStruct((NW, L), jnp.float32),
        mesh=mesh,
        scratch_types=[
            pltpu.VMEM((ROWS_PER_W,), jnp.int32),
            pltpu.VMEM((ROWS_PER_W,), jnp.int32),
            pltpu.VMEM((ROWS_PER_W,), jnp.float32),
            pltpu.VMEM((C,), jnp.float32),
            pltpu.VMEM((C,), jnp.float32),
            pltpu.VMEM((L,), jnp.float32),
            pltpu.SemaphoreType.DMA,
            pltpu.SemaphoreType.DMA,
        ],
        compiler_params=pltpu.CompilerParams(needs_layout_passes=False),
    )(flat, tflat)
    correct = out[:, 0].sum()
    valid = out[:, 1].sum()
    acc = correct / jnp.maximum(valid, 1.0)
    return jnp.where(valid == 0, jnp.float32(0.0), acc).astype(jnp.float32)


# speedup vs baseline: 3.5959x; 3.5959x over previous
import jax
import jax.numpy as jnp
from jax import lax
from jax.experimental import pallas as pl
from jax.experimental.pallas import tpu as pltpu
from jax.experimental.pallas import tpu_sc as plsc

TOPK = 5
IGN = -100
V = 100000
N = 1024
L = 16
NW = 32
G = 4
CWA = 7040
NA = 14
OB = CWA * NA
CWB = 1408
OT = OB + CWB
CWT = 32
CVA = CWA // L
CVB = CWB // L
UNROLL = 8


def _body(mat_hbm, targ_hbm, out_hbm, targv, tvbuf, accb, pcntb,
          bufa0, bufa1, bufb, tails, resv, sem_c, sem_w, sem_t):
    wid = lax.axis_index("s") * 2 + lax.axis_index("c")
    base = wid * (8 * G)

    pltpu.sync_copy(targ_hbm.at[pl.ds(base, 8 * G)], targv)

    lane = lax.iota(jnp.int32, L)
    zero16 = jnp.zeros((L,), jnp.int32)
    bufs = (bufa0, bufa1)

    def issue_windows(gnext):
        rs = pl.multiple_of(jnp.minimum(base + gnext * 8, N - 8), 8)
        for r in range(8):
            jn = jnp.minimum(gnext * 8 + r, 8 * G - 1)
            tvec = plsc.load_gather(targv, [jnp.full((L,), jn, jnp.int32)])
            tsc = jnp.clip(tvec[0], 0, V - 1)
            a128 = pl.multiple_of((tsc // 128) * 128, 128)
            pltpu.async_copy(mat_hbm.at[pl.ds(rs, 8), pl.ds(a128, 128)],
                             bufb.at[:, pl.ds(r * 128, 128)], sem_w)

    for gg in range(G):
        rs = base + gg * 8
        pltpu.async_copy(mat_hbm.at[pl.ds(rs, 8), pl.ds(OT, CWT)],
                         tails.at[gg], sem_t)
    issue_windows(0)
    pltpu.async_copy(mat_hbm.at[pl.ds(base, 8), pl.ds(0, CWA)], bufa0, sem_c)
    for gg in range(G):
        pltpu.make_async_copy(mat_hbm.at[pl.ds(0, 8), pl.ds(OT, CWT)],
                              tails.at[gg], sem_t).wait()

    def wait_w():
        for r in range(8):
            pltpu.make_async_copy(mat_hbm.at[pl.ds(0, 8), pl.ds(0, 128)],
                                  bufb.at[:, pl.ds(r * 128, 128)], sem_w).wait()

    def wait_a(buf):
        pltpu.make_async_copy(mat_hbm.at[pl.ds(0, 8), pl.ds(0, CWA)],
                              buf, sem_c).wait()

    def count_segment(buf, r, j, o, cv):
        tval16 = tvbuf[pl.ds(j * L, L)]
        tidx16 = plsc.load_gather(targv, [jnp.full((L,), j, jnp.int32)])
        tc = jnp.max(jnp.clip(tidx16, 0, V - 1))
        s = jnp.clip(tc - o, 0, cv * L)
        fs = s // L
        pcnt = pcntb[pl.ds(j * L, L)]
        acc = accb[pl.ds(j * L, L)]

        @plsc.parallel_loop(0, fs, unroll=UNROLL, carry=pcnt)
        def ge_loop(i, a):
            x = buf[r, pl.ds(i * L, L)]
            return a + plsc.all_reduce_population_count(x >= tval16)
        pcnt = ge_loop

        fm = jnp.minimum(fs, cv - 1)
        x = buf[r, pl.ds(fm * L, L)]
        posv = jnp.full((L,), o + fm * L, jnp.int32) + lane
        m = (x > tval16) | ((x == tval16) & (posv < tidx16))
        m = m & (jnp.full((L,), fs, jnp.int32) < cv)
        acc = acc + jnp.where(m, 1, 0).astype(jnp.int32)

        @plsc.parallel_loop(fs + 1, cv, unroll=UNROLL, carry=pcnt)
        def gt_loop(i, a):
            x = buf[r, pl.ds(i * L, L)]
            return a + plsc.all_reduce_population_count(x > tval16)
        pcnt = gt_loop

        pcntb[pl.ds(j * L, L)] = pcnt
        accb[pl.ds(j * L, L)] = acc

    def group_body(g, carry):
        cc, vc = carry
        rs_g = pl.multiple_of(base + g * 8, 8)

        wait_w()
        for r in range(8):
            j = g * 8 + r
            tidx16 = plsc.load_gather(targv, [jnp.full((L,), j, jnp.int32)])
            tcv = jnp.clip(tidx16, 0, V - 1)
            tval16 = plsc.load_gather(
                bufb, [jnp.full((L,), r, jnp.int32), r * 128 + (tcv % 128)])
            tvbuf[pl.ds(j * L, L)] = tval16
            pcntb[pl.ds(j * L, L)] = zero16
            accb[pl.ds(j * L, L)] = zero16

        def pair_body(p, _):
            for k in range(2):
                c = 2 * p + k
                wait_a(bufs[k])
                o_next = pl.multiple_of((c + 1) * CWA, 128)

                @pl.when(c < NA - 1)
                def _():
                    pltpu.async_copy(
                        mat_hbm.at[pl.ds(rs_g, 8), pl.ds(o_next, CWA)],
                        bufs[1 - k], sem_c)

                @pl.when(c == NA - 1)
                def _():
                    pltpu.async_copy(
                        mat_hbm.at[pl.ds(rs_g, 8), pl.ds(OB, CWB)],
                        bufb, sem_c)

                o = pl.multiple_of(c * CWA, 128)

                def rows_body(r, _):
                    count_segment(bufs[k], r, g * 8 + r, o, CVA)
                    return 0
                lax.fori_loop(0, 8, rows_body, 0)
            return 0
        lax.fori_loop(0, NA // 2, pair_body, 0)

        pltpu.make_async_copy(mat_hbm.at[pl.ds(0, 8), pl.ds(0, CWB)],
                              bufb, sem_c).wait()

        def rows_body_b(r, _):
            count_segment(bufb, r, g * 8 + r, OB, CVB)
            return 0
        lax.fori_loop(0, 8, rows_body_b, 0)

        issue_windows(g + 1)
        rs_n = pl.multiple_of(jnp.minimum(base + (g + 1) * 8, N - 8), 8)
        pltpu.async_copy(mat_hbm.at[pl.ds(rs_n, 8), pl.ds(0, CWA)],
                         bufa0, sem_c)

        def fin_body(r, carry2):
            cc2, vc2 = carry2
            j = g * 8 + r
            tval16 = tvbuf[pl.ds(j * L, L)]
            tidx16 = plsc.load_gather(targv, [jnp.full((L,), j, jnp.int32)])
            acc = accb[pl.ds(j * L, L)]
            for u in range(CWT // L):
                x = tails[g, r, pl.ds(u * L, L)]
                posv = jnp.full((L,), OT + u * L, jnp.int32) + lane
                m = (x > tval16) | ((x == tval16) & (posv < tidx16))
                acc = acc + jnp.where(m, 1, 0).astype(jnp.int32)
            rank = jnp.sum(acc) + jnp.max(pcntb[pl.ds(j * L, L)])
            t = jnp.max(tidx16)
            valid = t != IGN
            hit = (rank < TOPK) & valid
            cc2 = cc2 + jnp.where(hit, 1.0, 0.0)
            vc2 = vc2 + jnp.where(valid, 1.0, 0.0)
            return cc2, vc2
        cc, vc = lax.fori_loop(0, 8, fin_body, (cc, vc))
        return cc, vc

    cc, vc = lax.fori_loop(0, G, group_body,
                           (jnp.float32(0.0), jnp.float32(0.0)))

    wait_w()
    wait_a(bufa0)

    resv[...] = jnp.where(lane == 0, cc, jnp.where(lane == 1, vc, 0.0))
    pltpu.sync_copy(resv, out_hbm.at[wid])


@jax.jit
def kernel(logits, targets):
    mat = logits.reshape(N, V)
    tflat = targets.reshape(-1).astype(jnp.int32)
    mesh = plsc.VectorSubcoreMesh(core_axis_name="c", subcore_axis_name="s")
    out = pl.kernel(
        _body,
        out_type=jax.ShapeDtypeStruct((NW, L), jnp.float32),
        mesh=mesh,
        scratch_types=[
            pltpu.VMEM((8 * G,), jnp.int32),
            pltpu.VMEM((8 * G * L,), jnp.float32),
            pltpu.VMEM((8 * G * L,), jnp.int32),
            pltpu.VMEM((8 * G * L,), jnp.int32),
            pltpu.VMEM((8, CWA), jnp.float32),
            pltpu.VMEM((8, CWA), jnp.float32),
            pltpu.VMEM((8, CWB), jnp.float32),
            pltpu.VMEM((G, 8, CWT), jnp.float32),
            pltpu.VMEM((L,), jnp.float32),
            pltpu.SemaphoreType.DMA,
            pltpu.SemaphoreType.DMA,
            pltpu.SemaphoreType.DMA,
        ],
        compiler_params=pltpu.CompilerParams(needs_layout_passes=False),
    )(mat, tflat)
    correct = out[:, 0].sum()
    valid = out[:, 1].sum()
    acc = correct / jnp.maximum(valid, 1.0)
    return jnp.where(valid == 0, jnp.float32(0.0), acc).astype(jnp.float32)


# speedup vs baseline: 4.0637x; 1.1301x over previous
import jax
import jax.numpy as jnp
from jax import lax
from jax.experimental import pallas as pl
from jax.experimental.pallas import tpu as pltpu
from jax.experimental.pallas import tpu_sc as plsc

TOPK = 5
IGN = -100
V = 100000
N = 1024
L = 16
NW = 32
G = 4
OSC = 49920
CWA = 7040
NA = 7
ORG = OSC + CWA * NA
CRG = 768
OTL = ORG + CRG
CTL = V - OTL
CVA = CWA // L
CVR = CRG // L
CVT = CTL // L
UNROLL = 8


def _gather_body(mat_hbm, targ_hbm, tv_hbm, targv, wbuf, stage, sem_w):
    wid = lax.axis_index("s") * 2 + lax.axis_index("c")
    base = wid * (8 * G)
    pltpu.sync_copy(targ_hbm.at[pl.ds(base, 8 * G)], targv)

    for j in range(8 * G):
        v16 = targv[pl.ds((j // L) * L, L)]
        tj = jnp.clip(v16[j % L], 0, V - 1)
        a128 = pl.multiple_of((tj // 128) * 128, 128)
        rs = pl.multiple_of(base + (j // 8) * 8, 8)
        pltpu.async_copy(mat_hbm.at[pl.ds(rs, 8), pl.ds(a128, 128)],
                         wbuf.at[j], sem_w)
    for j in range(8 * G):
        pltpu.make_async_copy(mat_hbm.at[pl.ds(0, 8), pl.ds(0, 128)],
                              wbuf.at[j], sem_w).wait()
    for j in range(8 * G):
        tidx16 = plsc.load_gather(targv, [jnp.full((L,), j, jnp.int32)])
        tcv = jnp.clip(tidx16, 0, V - 1)
        tval16 = plsc.load_gather(
            wbuf, [jnp.full((L,), j, jnp.int32),
                   jnp.full((L,), j % 8, jnp.int32), tcv % 128])
        stage[pl.ds(j * L, L)] = tval16
    pltpu.sync_copy(stage, tv_hbm.at[pl.ds(base * L, 8 * G * L)])


def _count_body(mat_hbm, targ_hbm, tv_hbm, out_hbm, targv, tvv, accb, pcntb,
                bufa0, bufa1, bufr, buft, rblock, sem_c):
    wid = lax.axis_index("s") * 2 + lax.axis_index("c")
    base = wid * (8 * G)

    pltpu.sync_copy(targ_hbm.at[pl.ds(base, 8 * G)], targv)
    pltpu.sync_copy(tv_hbm.at[pl.ds(base * L, 8 * G * L)], tvv)

    lane = lax.iota(jnp.int32, L)
    zero16 = jnp.zeros((L,), jnp.int32)
    bufs = (bufa0, bufa1)
    for j in range(8 * G):
        accb[pl.ds(j * L, L)] = zero16
        pcntb[pl.ds(j * L, L)] = zero16

    pltpu.async_copy(mat_hbm.at[pl.ds(base, 8), pl.ds(OSC, CWA)], bufa0, sem_c)

    def wait_dma(buf, w):
        pltpu.make_async_copy(mat_hbm.at[pl.ds(0, 8), pl.ds(0, w)],
                              buf, sem_c).wait()

    def count_segment(buf, r, j, o, cv):
        tval16 = tvv[pl.ds(j * L, L)]
        tidx16 = plsc.load_gather(targv, [jnp.full((L,), j, jnp.int32)])
        tc = jnp.max(jnp.clip(tidx16, 0, V - 1))
        s = jnp.clip(tc - o, 0, cv * L)
        fs = s // L
        pcnt = pcntb[pl.ds(j * L, L)]
        acc = accb[pl.ds(j * L, L)]

        @plsc.parallel_loop(0, fs, unroll=UNROLL, carry=pcnt)
        def ge_loop(i, a):
            x = buf[r, pl.ds(i * L, L)]
            return a + plsc.all_reduce_population_count(x >= tval16)
        pcnt = ge_loop

        fm = jnp.minimum(fs, cv - 1)
        x = buf[r, pl.ds(fm * L, L)]
        posv = jnp.full((L,), o + fm * L, jnp.int32) + lane
        m = (x > tval16) | ((x == tval16) & (posv < tidx16))
        m = m & (jnp.full((L,), fs, jnp.int32) < cv)
        acc = acc + jnp.where(m, 1, 0).astype(jnp.int32)

        @plsc.parallel_loop(fs + 1, cv, unroll=UNROLL, carry=pcnt)
        def gt_loop(i, a):
            x = buf[r, pl.ds(i * L, L)]
            return a + plsc.all_reduce_population_count(x > tval16)
        pcnt = gt_loop

        pcntb[pl.ds(j * L, L)] = pcnt
        accb[pl.ds(j * L, L)] = acc

    def group_body(g, _):
        rs_g = pl.multiple_of(base + g * 8, 8)

        def pair_body(p, _2):
            for k in range(2):
                c = 2 * p + k
                wait_dma(bufs[k], CWA)
                o_next = pl.multiple_of(OSC + (c + 1) * CWA, 128)

                @pl.when(c < NA - 1)
                def _3():
                    pltpu.async_copy(
                        mat_hbm.at[pl.ds(rs_g, 8), pl.ds(o_next, CWA)],
                        bufs[1 - k], sem_c)

                o = pl.multiple_of(OSC + c * CWA, 128)

                def rows_body(r, _4):
                    count_segment(bufs[k], r, g * 8 + r, o, CVA)
                    return 0
                lax.fori_loop(0, 8, rows_body, 0)
            return 0
        lax.fori_loop(0, (NA - 1) // 2, pair_body, 0)

        wait_dma(bufs[0], CWA)
        pltpu.async_copy(mat_hbm.at[pl.ds(rs_g, 8), pl.ds(ORG, CRG)],
                         bufr, sem_c)
        pltpu.async_copy(mat_hbm.at[pl.ds(rs_g, 8), pl.ds(OTL, CTL)],
                         buft.at[0], sem_c)
        oc = pl.multiple_of(OSC + (NA - 1) * CWA, 128)

        def rows_body6(r, _5):
            count_segment(bufs[0], r, g * 8 + r, oc, CVA)
            return 0
        lax.fori_loop(0, 8, rows_body6, 0)

        wait_dma(bufr, CRG)
        pltpu.make_async_copy(mat_hbm.at[pl.ds(0, 8), pl.ds(OTL, CTL)],
                              buft.at[0], sem_c).wait()
        rs_n = pl.multiple_of(jnp.minimum(base + (g + 1) * 8, N - 8), 8)
        pltpu.async_copy(mat_hbm.at[pl.ds(rs_n, 8), pl.ds(OSC, CWA)],
                         bufa0, sem_c)

        def rows_bodyr(r, _6):
            count_segment(bufr, r, g * 8 + r, ORG, CVR)
            count_segment(buft.at[0], r, g * 8 + r, OTL, CVT)
            return 0
        lax.fori_loop(0, 8, rows_bodyr, 0)

        def fin_body(r, _7):
            j = g * 8 + r
            rank = (jnp.sum(accb[pl.ds(j * L, L)])
                    + jnp.max(pcntb[pl.ds(j * L, L)]))
            rblock[j, pl.ds(0, L)] = jnp.full((L,), rank, jnp.int32)
            return 0
        lax.fori_loop(0, 8, fin_body, 0)
        return 0

    lax.fori_loop(0, G, group_body, 0)
    wait_dma(bufa0, CWA)
    pltpu.sync_copy(rblock, out_hbm.at[pl.ds(base, 8 * G), :])


def _tc_body(x_ref, tv_ref, tg_ref, o_ref):
    x = x_ref[...]
    tv = tv_ref[...]
    tg = tg_ref[...]
    ci = lax.broadcasted_iota(jnp.int32, x.shape, 1)
    m = (x > tv) | ((x == tv) & (ci < tg))
    o_ref[...] = jnp.sum(m.astype(jnp.int32), axis=1, keepdims=True)


@jax.jit
def kernel(logits, targets):
    mat = logits.reshape(N, V)
    tflat = targets.reshape(-1).astype(jnp.int32)
    mesh = plsc.VectorSubcoreMesh(core_axis_name="c", subcore_axis_name="s")
    scp = pltpu.CompilerParams(needs_layout_passes=False)

    tvals = pl.kernel(
        _gather_body,
        out_type=jax.ShapeDtypeStruct((N * L,), jnp.float32),
        mesh=mesh,
        scratch_types=[
            pltpu.VMEM((8 * G,), jnp.int32),
            pltpu.VMEM((8 * G, 8, 128), jnp.float32),
            pltpu.VMEM((8 * G * L,), jnp.float32),
            pltpu.SemaphoreType.DMA,
        ],
        compiler_params=scp,
    )(mat, tflat)

    rank_sc = pl.kernel(
        _count_body,
        out_type=jax.ShapeDtypeStruct((N, L), jnp.int32),
        mesh=mesh,
        scratch_types=[
            pltpu.VMEM((8 * G,), jnp.int32),
            pltpu.VMEM((8 * G * L,), jnp.float32),
            pltpu.VMEM((8 * G * L,), jnp.int32),
            pltpu.VMEM((8 * G * L,), jnp.int32),
            pltpu.VMEM((8, CWA), jnp.float32),
            pltpu.VMEM((8, CWA), jnp.float32),
            pltpu.VMEM((8, CRG), jnp.float32),
            pltpu.VMEM((1, 8, CTL), jnp.float32),
            pltpu.VMEM((8 * G, L), jnp.int32),
            pltpu.SemaphoreType.DMA,
        ],
        compiler_params=scp,
    )(mat, tflat, tvals)

    tv2 = tvals.reshape(N, L)[:, :1]
    tg2 = tflat[:, None]
    rank_tc = pl.pallas_call(
        _tc_body,
        grid=(N // 8,),
        in_specs=[
            pl.BlockSpec((8, OSC), lambda i: (i, 0)),
            pl.BlockSpec((8, 1), lambda i: (i, 0)),
            pl.BlockSpec((8, 1), lambda i: (i, 0)),
        ],
        out_specs=pl.BlockSpec((8, 1), lambda i: (i, 0)),
        out_shape=jax.ShapeDtypeStruct((N, 1), jnp.int32),
    )(mat, tv2, tg2)

    rank = rank_sc[:, 0] + rank_tc[:, 0]
    valid = tflat != IGN
    hit = (rank < TOPK) & valid
    correct = hit.sum().astype(jnp.float32)
    vcnt = valid.sum().astype(jnp.float32)
    acc = correct / jnp.maximum(vcnt, 1.0)
    return jnp.where(vcnt == 0, jnp.float32(0.0), acc).astype(jnp.float32)


# speedup vs baseline: 4.2364x; 1.0425x over previous
import jax
import jax.numpy as jnp
from jax import lax
from jax.experimental import pallas as pl
from jax.experimental.pallas import tpu as pltpu
from jax.experimental.pallas import tpu_sc as plsc

TOPK = 5
IGN = -100
V = 100000
N = 1024
L = 16
NW = 32
G = 4
OSC = 42624
CWA = 7040
NA = 8
ORG = OSC + CWA * NA
CRG = 1024
OTL = ORG + CRG
CTL = V - OTL
CVA = CWA // L
CVR = CRG // L
CVT = CTL // L
UNROLL = 8


def _gather_body(mat_hbm, targ_hbm, tv_hbm, targv, wbuf, stage, sem_w):
    wid = lax.axis_index("s") * 2 + lax.axis_index("c")
    base = wid * (8 * G)
    pltpu.sync_copy(targ_hbm.at[pl.ds(base, 8 * G)], targv)

    for j in range(8 * G):
        v16 = targv[pl.ds((j // L) * L, L)]
        tj = jnp.clip(v16[j % L], 0, V - 1)
        a128 = pl.multiple_of((tj // 128) * 128, 128)
        rs = pl.multiple_of(base + (j // 8) * 8, 8)
        pltpu.async_copy(mat_hbm.at[pl.ds(rs, 8), pl.ds(a128, 128)],
                         wbuf.at[j], sem_w)
    for j in range(8 * G):
        pltpu.make_async_copy(mat_hbm.at[pl.ds(0, 8), pl.ds(0, 128)],
                              wbuf.at[j], sem_w).wait()
    for j in range(8 * G):
        tidx16 = plsc.load_gather(targv, [jnp.full((L,), j, jnp.int32)])
        tcv = jnp.clip(tidx16, 0, V - 1)
        tval16 = plsc.load_gather(
            wbuf, [jnp.full((L,), j, jnp.int32),
                   jnp.full((L,), j % 8, jnp.int32), tcv % 128])
        stage[pl.ds(j * L, L)] = tval16
    pltpu.sync_copy(stage, tv_hbm.at[pl.ds(base * L, 8 * G * L)])


def _count_body(mat_hbm, targ_hbm, tv_hbm, out_hbm, targv, tvv, accb, pcntb,
                bufa0, bufa1, bufr, buft, rblock, sem_c):
    wid = lax.axis_index("s") * 2 + lax.axis_index("c")
    base = wid * (8 * G)

    pltpu.sync_copy(targ_hbm.at[pl.ds(base, 8 * G)], targv)
    pltpu.sync_copy(tv_hbm.at[pl.ds(base * L, 8 * G * L)], tvv)

    lane = lax.iota(jnp.int32, L)
    zero16 = jnp.zeros((L,), jnp.int32)
    bufs = (bufa0, bufa1)
    for j in range(8 * G):
        accb[pl.ds(j * L, L)] = zero16
        pcntb[pl.ds(j * L, L)] = zero16

    pltpu.async_copy(mat_hbm.at[pl.ds(base, 8), pl.ds(OSC, CWA)], bufa0, sem_c)

    def wait_dma(buf, w):
        pltpu.make_async_copy(mat_hbm.at[pl.ds(0, 8), pl.ds(0, w)],
                              buf, sem_c).wait()

    def count_segment(buf, r, j, o, cv):
        tval16 = tvv[pl.ds(j * L, L)]
        tidx16 = plsc.load_gather(targv, [jnp.full((L,), j, jnp.int32)])
        tc = jnp.max(jnp.clip(tidx16, 0, V - 1))
        s = jnp.clip(tc - o, 0, cv * L)
        fs = s // L
        pcnt = pcntb[pl.ds(j * L, L)]
        acc = accb[pl.ds(j * L, L)]

        @plsc.parallel_loop(0, fs, unroll=UNROLL, carry=pcnt)
        def ge_loop(i, a):
            x = buf[r, pl.ds(i * L, L)]
            return a + plsc.all_reduce_population_count(x >= tval16)
        pcnt = ge_loop

        fm = jnp.minimum(fs, cv - 1)
        x = buf[r, pl.ds(fm * L, L)]
        posv = jnp.full((L,), o + fm * L, jnp.int32) + lane
        m = (x > tval16) | ((x == tval16) & (posv < tidx16))
        m = m & (jnp.full((L,), fs, jnp.int32) < cv)
        acc = acc + jnp.where(m, 1, 0).astype(jnp.int32)

        @plsc.parallel_loop(fs + 1, cv, unroll=UNROLL, carry=pcnt)
        def gt_loop(i, a):
            x = buf[r, pl.ds(i * L, L)]
            return a + plsc.all_reduce_population_count(x > tval16)
        pcnt = gt_loop

        pcntb[pl.ds(j * L, L)] = pcnt
        accb[pl.ds(j * L, L)] = acc

    def group_body(g, _):
        rs_g = pl.multiple_of(base + g * 8, 8)

        def pair_body(p, _2):
            for k in range(2):
                c = 2 * p + k
                wait_dma(bufs[k], CWA)
                o_next = pl.multiple_of(OSC + (c + 1) * CWA, 128)

                @pl.when(c < NA - 1)
                def _3():
                    pltpu.async_copy(
                        mat_hbm.at[pl.ds(rs_g, 8), pl.ds(o_next, CWA)],
                        bufs[1 - k], sem_c)

                @pl.when(c == NA - 1)
                def _3b():
                    pltpu.async_copy(
                        mat_hbm.at[pl.ds(rs_g, 8), pl.ds(ORG, CRG)],
                        bufr, sem_c)
                    pltpu.async_copy(
                        mat_hbm.at[pl.ds(rs_g, 8), pl.ds(OTL, CTL)],
                        buft.at[0], sem_c)

                o = pl.multiple_of(OSC + c * CWA, 128)

                def rows_body(r, _4):
                    count_segment(bufs[k], r, g * 8 + r, o, CVA)
                    return 0
                lax.fori_loop(0, 8, rows_body, 0)
            return 0
        lax.fori_loop(0, NA // 2, pair_body, 0)

        wait_dma(bufr, CRG)
        pltpu.make_async_copy(mat_hbm.at[pl.ds(0, 8), pl.ds(OTL, CTL)],
                              buft.at[0], sem_c).wait()
        rs_n = pl.multiple_of(jnp.minimum(base + (g + 1) * 8, N - 8), 8)
        pltpu.async_copy(mat_hbm.at[pl.ds(rs_n, 8), pl.ds(OSC, CWA)],
                         bufa0, sem_c)

        def rows_bodyr(r, _6):
            count_segment(bufr, r, g * 8 + r, ORG, CVR)
            count_segment(buft.at[0], r, g * 8 + r, OTL, CVT)
            return 0
        lax.fori_loop(0, 8, rows_bodyr, 0)

        def fin_body(r, _7):
            j = g * 8 + r
            rank = (jnp.sum(accb[pl.ds(j * L, L)])
                    + jnp.max(pcntb[pl.ds(j * L, L)]))
            rblock[j, pl.ds(0, L)] = jnp.full((L,), rank, jnp.int32)
            return 0
        lax.fori_loop(0, 8, fin_body, 0)
        return 0

    lax.fori_loop(0, G, group_body, 0)
    wait_dma(bufa0, CWA)
    pltpu.sync_copy(rblock, out_hbm.at[pl.ds(base, 8 * G), :])


def _tc_body(x_ref, tv_ref, tg_ref, o_ref):
    x = x_ref[...]
    tv = tv_ref[...]
    tg = tg_ref[...]
    ci = lax.broadcasted_iota(jnp.int32, x.shape, 1)
    m = (x > tv) | ((x == tv) & (ci < tg))
    o_ref[...] = jnp.sum(m.astype(jnp.int32), axis=1, keepdims=True)


@jax.jit
def kernel(logits, targets):
    mat = logits.reshape(N, V)
    tflat = targets.reshape(-1).astype(jnp.int32)
    mesh = plsc.VectorSubcoreMesh(core_axis_name="c", subcore_axis_name="s")
    scp = pltpu.CompilerParams(needs_layout_passes=False)

    tvals = pl.kernel(
        _gather_body,
        out_type=jax.ShapeDtypeStruct((N * L,), jnp.float32),
        mesh=mesh,
        scratch_types=[
            pltpu.VMEM((8 * G,), jnp.int32),
            pltpu.VMEM((8 * G, 8, 128), jnp.float32),
            pltpu.VMEM((8 * G * L,), jnp.float32),
            pltpu.SemaphoreType.DMA,
        ],
        compiler_params=scp,
    )(mat, tflat)

    rank_sc = pl.kernel(
        _count_body,
        out_type=jax.ShapeDtypeStruct((N, L), jnp.int32),
        mesh=mesh,
        scratch_types=[
            pltpu.VMEM((8 * G,), jnp.int32),
            pltpu.VMEM((8 * G * L,), jnp.float32),
            pltpu.VMEM((8 * G * L,), jnp.int32),
            pltpu.VMEM((8 * G * L,), jnp.int32),
            pltpu.VMEM((8, CWA), jnp.float32),
            pltpu.VMEM((8, CWA), jnp.float32),
            pltpu.VMEM((8, CRG), jnp.float32),
            pltpu.VMEM((1, 8, CTL), jnp.float32),
            pltpu.VMEM((8 * G, L), jnp.int32),
            pltpu.SemaphoreType.DMA,
        ],
        compiler_params=scp,
    )(mat, tflat, tvals)

    tv2 = tvals.reshape(N, L)[:, :1]
    tg2 = tflat[:, None]
    rank_tc = pl.pallas_call(
        _tc_body,
        grid=(N // 8,),
        in_specs=[
            pl.BlockSpec((8, OSC), lambda i: (i, 0)),
            pl.BlockSpec((8, 1), lambda i: (i, 0)),
            pl.BlockSpec((8, 1), lambda i: (i, 0)),
        ],
        out_specs=pl.BlockSpec((8, 1), lambda i: (i, 0)),
        out_shape=jax.ShapeDtypeStruct((N, 1), jnp.int32),
    )(mat, tv2, tg2)

    rank = rank_sc[:, 0] + rank_tc[:, 0]
    valid = tflat != IGN
    hit = (rank < TOPK) & valid
    correct = hit.sum().astype(jnp.float32)
    vcnt = valid.sum().astype(jnp.float32)
    acc = correct / jnp.maximum(vcnt, 1.0)
    return jnp.where(vcnt == 0, jnp.float32(0.0), acc).astype(jnp.float32)


# speedup vs baseline: 4.3420x; 1.0249x over previous
import jax
import jax.numpy as jnp
from jax import lax
from jax.experimental import pallas as pl
from jax.experimental.pallas import tpu as pltpu
from jax.experimental.pallas import tpu_sc as plsc

TOPK = 5
IGN = -100
V = 100000
N = 1024
L = 16
NW = 32
G = 4
OSC = 39936
CWA = 5888
NA = 10
ORG = OSC + CWA * NA
CRG = 1152
OTL = ORG + CRG
CTL = V - OTL
CVA = CWA // L
CVR = CRG // L
CVT = CTL // L
UNROLL = 8


def _gather_body(mat_hbm, targ_hbm, tv_hbm, targv, wbuf, stage, sem_w):
    wid = lax.axis_index("s") * 2 + lax.axis_index("c")
    base = wid * (8 * G)
    pltpu.sync_copy(targ_hbm.at[pl.ds(base, 8 * G)], targv)

    for j in range(8 * G):
        v16 = targv[pl.ds((j // L) * L, L)]
        tj = jnp.clip(v16[j % L], 0, V - 1)
        a128 = pl.multiple_of((tj // 128) * 128, 128)
        rs = pl.multiple_of(base + (j // 8) * 8, 8)
        pltpu.async_copy(mat_hbm.at[pl.ds(rs, 8), pl.ds(a128, 128)],
                         wbuf.at[j], sem_w)
    for j in range(8 * G):
        pltpu.make_async_copy(mat_hbm.at[pl.ds(0, 8), pl.ds(0, 128)],
                              wbuf.at[j], sem_w).wait()
    for j in range(8 * G):
        tidx16 = plsc.load_gather(targv, [jnp.full((L,), j, jnp.int32)])
        tcv = jnp.clip(tidx16, 0, V - 1)
        tval16 = plsc.load_gather(
            wbuf, [jnp.full((L,), j, jnp.int32),
                   jnp.full((L,), j % 8, jnp.int32), tcv % 128])
        stage[pl.ds(j * L, L)] = tval16
    pltpu.sync_copy(stage, tv_hbm.at[pl.ds(base * L, 8 * G * L)])


def _count_body(mat_hbm, targ_hbm, tv_hbm, out_hbm, targv, tvv, accb, pcntb,
                bufa0, bufa1, bufr, buft, rblock, sem_c):
    wid = lax.axis_index("s") * 2 + lax.axis_index("c")
    base = wid * (8 * G)

    pltpu.sync_copy(targ_hbm.at[pl.ds(base, 8 * G)], targv)
    pltpu.sync_copy(tv_hbm.at[pl.ds(base * L, 8 * G * L)], tvv)

    lane = lax.iota(jnp.int32, L)
    zero16 = jnp.zeros((L,), jnp.int32)
    bufs = (bufa0, bufa1)
    for j in range(8 * G):
        accb[pl.ds(j * L, L)] = zero16
        pcntb[pl.ds(j * L, L)] = zero16

    pltpu.async_copy(mat_hbm.at[pl.ds(base, 8), pl.ds(OSC, CWA)], bufa0, sem_c)

    def wait_dma(buf, w):
        pltpu.make_async_copy(mat_hbm.at[pl.ds(0, 8), pl.ds(0, w)],
                              buf, sem_c).wait()

    def count_segment(buf, r, j, o, cv):
        tval16 = tvv[pl.ds(j * L, L)]
        tidx16 = plsc.load_gather(targv, [jnp.full((L,), j, jnp.int32)])
        tc = jnp.max(jnp.clip(tidx16, 0, V - 1))
        s = jnp.clip(tc - o, 0, cv * L)
        fs = s // L
        pcnt = pcntb[pl.ds(j * L, L)]
        acc = accb[pl.ds(j * L, L)]

        @plsc.parallel_loop(0, fs, unroll=UNROLL, carry=pcnt)
        def ge_loop(i, a):
            x = buf[r, pl.ds(i * L, L)]
            return a + plsc.all_reduce_population_count(x >= tval16)
        pcnt = ge_loop

        fm = jnp.minimum(fs, cv - 1)
        x = buf[r, pl.ds(fm * L, L)]
        posv = jnp.full((L,), o + fm * L, jnp.int32) + lane
        m = (x > tval16) | ((x == tval16) & (posv < tidx16))
        m = m & (jnp.full((L,), fs, jnp.int32) < cv)
        acc = acc + jnp.where(m, 1, 0).astype(jnp.int32)

        @plsc.parallel_loop(fs + 1, cv, unroll=UNROLL, carry=pcnt)
        def gt_loop(i, a):
            x = buf[r, pl.ds(i * L, L)]
            return a + plsc.all_reduce_population_count(x > tval16)
        pcnt = gt_loop

        pcntb[pl.ds(j * L, L)] = pcnt
        accb[pl.ds(j * L, L)] = acc

    def group_body(g, _):
        rs_g = pl.multiple_of(base + g * 8, 8)

        def pair_body(p, _2):
            for k in range(2):
                c = 2 * p + k
                wait_dma(bufs[k], CWA)
                o_next = pl.multiple_of(OSC + (c + 1) * CWA, 128)

                @pl.when(c < NA - 1)
                def _3():
                    pltpu.async_copy(
                        mat_hbm.at[pl.ds(rs_g, 8), pl.ds(o_next, CWA)],
                        bufs[1 - k], sem_c)

                @pl.when(c == NA - 1)
                def _3b():
                    pltpu.async_copy(
                        mat_hbm.at[pl.ds(rs_g, 8), pl.ds(ORG, CRG)],
                        bufr, sem_c)
                    pltpu.async_copy(
                        mat_hbm.at[pl.ds(rs_g, 8), pl.ds(OTL, CTL)],
                        buft.at[0], sem_c)

                o = pl.multiple_of(OSC + c * CWA, 128)

                def rows_body(r, _4):
                    count_segment(bufs[k], r, g * 8 + r, o, CVA)
                    return 0
                lax.fori_loop(0, 8, rows_body, 0)
            return 0
        lax.fori_loop(0, NA // 2, pair_body, 0)

        wait_dma(bufr, CRG)
        pltpu.make_async_copy(mat_hbm.at[pl.ds(0, 8), pl.ds(OTL, CTL)],
                              buft.at[0], sem_c).wait()
        rs_n = pl.multiple_of(jnp.minimum(base + (g + 1) * 8, N - 8), 8)
        pltpu.async_copy(mat_hbm.at[pl.ds(rs_n, 8), pl.ds(OSC, CWA)],
                         bufa0, sem_c)

        def rows_bodyr(r, _6):
            count_segment(bufr, r, g * 8 + r, ORG, CVR)
            count_segment(buft.at[0], r, g * 8 + r, OTL, CVT)
            return 0
        lax.fori_loop(0, 8, rows_bodyr, 0)

        def fin_body(r, _7):
            j = g * 8 + r
            rank = (jnp.sum(accb[pl.ds(j * L, L)])
                    + jnp.max(pcntb[pl.ds(j * L, L)]))
            rblock[j, pl.ds(0, L)] = jnp.full((L,), rank, jnp.int32)
            return 0
        lax.fori_loop(0, 8, fin_body, 0)
        return 0

    lax.fori_loop(0, G, group_body, 0)
    wait_dma(bufa0, CWA)
    pltpu.sync_copy(rblock, out_hbm.at[pl.ds(base, 8 * G), :])


def _tc_body(x_ref, tv_ref, tg_ref, o_ref):
    x = x_ref[...]
    tv = tv_ref[...]
    tg = tg_ref[...]
    ci = lax.broadcasted_iota(jnp.int32, x.shape, 1)
    m = (x > tv) | ((x == tv) & (ci < tg))
    o_ref[...] = jnp.sum(m.astype(jnp.int32), axis=1, keepdims=True)


@jax.jit
def kernel(logits, targets):
    mat = logits.reshape(N, V)
    tflat = targets.reshape(-1).astype(jnp.int32)
    mesh = plsc.VectorSubcoreMesh(core_axis_name="c", subcore_axis_name="s")
    scp = pltpu.CompilerParams(needs_layout_passes=False)

    tvals = pl.kernel(
        _gather_body,
        out_type=jax.ShapeDtypeStruct((N * L,), jnp.float32),
        mesh=mesh,
        scratch_types=[
            pltpu.VMEM((8 * G,), jnp.int32),
            pltpu.VMEM((8 * G, 8, 128), jnp.float32),
            pltpu.VMEM((8 * G * L,), jnp.float32),
            pltpu.SemaphoreType.DMA,
        ],
        compiler_params=scp,
    )(mat, tflat)

    rank_sc = pl.kernel(
        _count_body,
        out_type=jax.ShapeDtypeStruct((N, L), jnp.int32),
        mesh=mesh,
        scratch_types=[
            pltpu.VMEM((8 * G,), jnp.int32),
            pltpu.VMEM((8 * G * L,), jnp.float32),
            pltpu.VMEM((8 * G * L,), jnp.int32),
            pltpu.VMEM((8 * G * L,), jnp.int32),
            pltpu.VMEM((8, CWA), jnp.float32),
            pltpu.VMEM((8, CWA), jnp.float32),
            pltpu.VMEM((8, CRG), jnp.float32),
            pltpu.VMEM((1, 8, CTL), jnp.float32),
            pltpu.VMEM((8 * G, L), jnp.int32),
            pltpu.SemaphoreType.DMA,
        ],
        compiler_params=scp,
    )(mat, tflat, tvals)

    tv2 = tvals.reshape(N, L)[:, :1]
    tg2 = tflat[:, None]
    rank_tc = pl.pallas_call(
        _tc_body,
        grid=(N // 32,),
        in_specs=[
            pl.BlockSpec((32, OSC), lambda i: (i, 0)),
            pl.BlockSpec((32, 1), lambda i: (i, 0)),
            pl.BlockSpec((32, 1), lambda i: (i, 0)),
        ],
        out_specs=pl.BlockSpec((32, 1), lambda i: (i, 0)),
        out_shape=jax.ShapeDtypeStruct((N, 1), jnp.int32),
    )(mat, tv2, tg2)

    rank = rank_sc[:, 0] + rank_tc[:, 0]
    valid = tflat != IGN
    hit = (rank < TOPK) & valid
    correct = hit.sum().astype(jnp.float32)
    vcnt = valid.sum().astype(jnp.float32)
    acc = correct / jnp.maximum(vcnt, 1.0)
    return jnp.where(vcnt == 0, jnp.float32(0.0), acc).astype(jnp.float32)


# speedup vs baseline: 4.5973x; 1.0588x over previous
import jax
import jax.numpy as jnp
from jax import lax
from jax.experimental import pallas as pl
from jax.experimental.pallas import tpu as pltpu
from jax.experimental.pallas import tpu_sc as plsc

TOPK = 5
IGN = -100
V = 100000
N = 1024
L = 16
NW = 32
G = 4
OSC = 49280
CWA = 6272
NA = 8
ORG = OSC + CWA * NA
CRG = 512
OTL = ORG + CRG
CTL = V - OTL
CVA = CWA // L
CVR = CRG // L
CVT = CTL // L
UNROLL = 8


def _gather_body(mat_hbm, targ_hbm, tv_hbm, targv, wbuf, stage, sem_w):
    wid = lax.axis_index("s") * 2 + lax.axis_index("c")
    base = wid * (8 * G)
    pltpu.sync_copy(targ_hbm.at[pl.ds(base, 8 * G)], targv)

    for j in range(8 * G):
        v16 = targv[pl.ds((j // L) * L, L)]
        tj = jnp.clip(v16[j % L], 0, V - 1)
        a128 = pl.multiple_of((tj // 128) * 128, 128)
        rs = pl.multiple_of(base + (j // 8) * 8, 8)
        pltpu.async_copy(mat_hbm.at[pl.ds(rs, 8), pl.ds(a128, 128)],
                         wbuf.at[j], sem_w)
    for j in range(8 * G):
        pltpu.make_async_copy(mat_hbm.at[pl.ds(0, 8), pl.ds(0, 128)],
                              wbuf.at[j], sem_w).wait()
    for j in range(8 * G):
        tidx16 = plsc.load_gather(targv, [jnp.full((L,), j, jnp.int32)])
        tcv = jnp.clip(tidx16, 0, V - 1)
        tval16 = plsc.load_gather(
            wbuf, [jnp.full((L,), j, jnp.int32),
                   jnp.full((L,), j % 8, jnp.int32), tcv % 128])
        stage[pl.ds(j * L, L)] = tval16
    pltpu.sync_copy(stage, tv_hbm.at[pl.ds(base * L, 8 * G * L)])


def _count_body(mat_hbm, targ_hbm, tv_hbm, out_hbm, targv, tvv, accb, pcntb,
                bufa0, bufa1, bufr, buft, rblock, sem_c):
    wid = lax.axis_index("s") * 2 + lax.axis_index("c")
    base = wid * (8 * G)

    pltpu.sync_copy(targ_hbm.at[pl.ds(base, 8 * G)], targv)
    pltpu.sync_copy(tv_hbm.at[pl.ds(base * L, 8 * G * L)], tvv)

    lane = lax.iota(jnp.int32, L)
    zero16 = jnp.zeros((L,), jnp.int32)
    bufs = (bufa0, bufa1)
    for j in range(8 * G):
        accb[pl.ds(j * L, L)] = zero16
        pcntb[pl.ds(j * L, L)] = zero16

    pltpu.async_copy(mat_hbm.at[pl.ds(base, 8), pl.ds(OSC, CWA)], bufa0, sem_c)

    def wait_dma(buf, w):
        pltpu.make_async_copy(mat_hbm.at[pl.ds(0, 8), pl.ds(0, w)],
                              buf, sem_c).wait()

    def count_segment(buf, r, j, o, cv):
        tval16 = tvv[pl.ds(j * L, L)]
        tidx16 = plsc.load_gather(targv, [jnp.full((L,), j, jnp.int32)])
        tc = jnp.max(jnp.clip(tidx16, 0, V - 1))
        s = jnp.clip(tc - o, 0, cv * L)
        fs = s // L
        pcnt = pcntb[pl.ds(j * L, L)]
        acc = accb[pl.ds(j * L, L)]

        @plsc.parallel_loop(0, fs, unroll=UNROLL, carry=pcnt)
        def ge_loop(i, a):
            x = buf[r, pl.ds(i * L, L)]
            return a + plsc.all_reduce_population_count(x >= tval16)
        pcnt = ge_loop

        fm = jnp.minimum(fs, cv - 1)
        x = buf[r, pl.ds(fm * L, L)]
        posv = jnp.full((L,), o + fm * L, jnp.int32) + lane
        m = (x > tval16) | ((x == tval16) & (posv < tidx16))
        m = m & (jnp.full((L,), fs, jnp.int32) < cv)
        acc = acc + jnp.where(m, 1, 0).astype(jnp.int32)

        @plsc.parallel_loop(fs + 1, cv, unroll=UNROLL, carry=pcnt)
        def gt_loop(i, a):
            x = buf[r, pl.ds(i * L, L)]
            return a + plsc.all_reduce_population_count(x > tval16)
        pcnt = gt_loop

        pcntb[pl.ds(j * L, L)] = pcnt
        accb[pl.ds(j * L, L)] = acc

    def group_body(g, _):
        rs_g = pl.multiple_of(base + g * 8, 8)

        def pair_body(p, _2):
            for k in range(2):
                c = 2 * p + k
                wait_dma(bufs[k], CWA)
                o_next = pl.multiple_of(OSC + (c + 1) * CWA, 128)

                @pl.when(c < NA - 1)
                def _3():
                    pltpu.async_copy(
                        mat_hbm.at[pl.ds(rs_g, 8), pl.ds(o_next, CWA)],
                        bufs[1 - k], sem_c)

                @pl.when(c == NA - 1)
                def _3b():
                    pltpu.async_copy(
                        mat_hbm.at[pl.ds(rs_g, 8), pl.ds(ORG, CRG)],
                        bufr, sem_c)
                    pltpu.async_copy(
                        mat_hbm.at[pl.ds(rs_g, 8), pl.ds(OTL, CTL)],
                        buft.at[0], sem_c)

                o = pl.multiple_of(OSC + c * CWA, 128)

                def rows_body(r, _4):
                    count_segment(bufs[k], r, g * 8 + r, o, CVA)
                    return 0
                lax.fori_loop(0, 8, rows_body, 0)
            return 0
        lax.fori_loop(0, NA // 2, pair_body, 0)

        wait_dma(bufr, CRG)
        pltpu.make_async_copy(mat_hbm.at[pl.ds(0, 8), pl.ds(OTL, CTL)],
                              buft.at[0], sem_c).wait()
        rs_n = pl.multiple_of(jnp.minimum(base + (g + 1) * 8, N - 8), 8)
        pltpu.async_copy(mat_hbm.at[pl.ds(rs_n, 8), pl.ds(OSC, CWA)],
                         bufa0, sem_c)

        def rows_bodyr(r, _6):
            count_segment(bufr, r, g * 8 + r, ORG, CVR)
            count_segment(buft.at[0], r, g * 8 + r, OTL, CVT)
            return 0
        lax.fori_loop(0, 8, rows_bodyr, 0)

        def fin_body(r, _7):
            j = g * 8 + r
            rank = (jnp.sum(accb[pl.ds(j * L, L)])
                    + jnp.max(pcntb[pl.ds(j * L, L)]))
            rblock[j, pl.ds(0, L)] = jnp.full((L,), rank, jnp.int32)
            return 0
        lax.fori_loop(0, 8, fin_body, 0)
        return 0

    lax.fori_loop(0, G, group_body, 0)
    wait_dma(bufa0, CWA)
    pltpu.sync_copy(rblock, out_hbm.at[pl.ds(base, 8 * G), :])


def _tc_body(x_ref, tv_ref, tg_ref, o_ref):
    x = x_ref[...]
    tv = tv_ref[...]
    tg = tg_ref[...]
    ci = lax.broadcasted_iota(jnp.int32, x.shape, 1)
    m = (x > tv) | ((x == tv) & (ci < tg))
    o_ref[...] = jnp.sum(m.astype(jnp.int32), axis=1, keepdims=True)


@jax.jit
def kernel(logits, targets):
    mat = logits.reshape(N, V)
    tflat = targets.reshape(-1).astype(jnp.int32)
    mesh = plsc.VectorSubcoreMesh(core_axis_name="c", subcore_axis_name="s")
    scp = pltpu.CompilerParams(needs_layout_passes=False)

    tvals = pl.kernel(
        _gather_body,
        out_type=jax.ShapeDtypeStruct((N * L,), jnp.float32),
        mesh=mesh,
        scratch_types=[
            pltpu.VMEM((8 * G,), jnp.int32),
            pltpu.VMEM((8 * G, 8, 128), jnp.float32),
            pltpu.VMEM((8 * G * L,), jnp.float32),
            pltpu.SemaphoreType.DMA,
        ],
        compiler_params=scp,
    )(mat, tflat)

    rank_sc = pl.kernel(
        _count_body,
        out_type=jax.ShapeDtypeStruct((N, L), jnp.int32),
        mesh=mesh,
        scratch_types=[
            pltpu.VMEM((8 * G,), jnp.int32),
            pltpu.VMEM((8 * G * L,), jnp.float32),
            pltpu.VMEM((8 * G * L,), jnp.int32),
            pltpu.VMEM((8 * G * L,), jnp.int32),
            pltpu.VMEM((8, CWA), jnp.float32),
            pltpu.VMEM((8, CWA), jnp.float32),
            pltpu.VMEM((8, CRG), jnp.float32),
            pltpu.VMEM((1, 8, CTL), jnp.float32),
            pltpu.VMEM((8 * G, L), jnp.int32),
            pltpu.SemaphoreType.DMA,
        ],
        compiler_params=scp,
    )(mat, tflat, tvals)

    tv2 = tvals.reshape(N, L)[:, :1]
    tg2 = tflat[:, None]
    rank_tc = pl.pallas_call(
        _tc_body,
        grid=(N // 32,),
        in_specs=[
            pl.BlockSpec((32, OSC), lambda i: (i, 0)),
            pl.BlockSpec((32, 1), lambda i: (i, 0)),
            pl.BlockSpec((32, 1), lambda i: (i, 0)),
        ],
        out_specs=pl.BlockSpec((32, 1), lambda i: (i, 0)),
        out_shape=jax.ShapeDtypeStruct((N, 1), jnp.int32),
    )(mat, tv2, tg2)

    rank = rank_sc[:, 0] + rank_tc[:, 0]
    valid = tflat != IGN
    hit = (rank < TOPK) & valid
    correct = hit.sum().astype(jnp.float32)
    vcnt = valid.sum().astype(jnp.float32)
    acc = correct / jnp.maximum(vcnt, 1.0)
    return jnp.where(vcnt == 0, jnp.float32(0.0), acc).astype(jnp.float32)
